# Initial kernel scaffold; baseline (speedup 1.0000x reference)
import jax
import jax.numpy as jnp
from jax.experimental import pallas as pl


def kernel(x, Wg, bg, W1, b1, W2, b2):
    raise NotImplementedError("write your pallas kernel here")


# speedup vs baseline: 1.0788x; 1.0788x over previous
import functools

import jax
import jax.numpy as jnp
from jax.experimental import pallas as pl
from jax.experimental.pallas import tpu as pltpu

_E = 8
_EPAD = 128
_FBLK = 512


def _router_body(x_ref, wg_ref, bg_ref, logits_ref, gates_ref):
    x = x_ref[...]
    lg = jnp.dot(x, wg_ref[...], preferred_element_type=jnp.float32)
    lg = lg + bg_ref[...]
    col = jax.lax.broadcasted_iota(jnp.int32, lg.shape, 1)
    neg = jnp.float32(-jnp.inf)
    m1 = jnp.max(lg, axis=1, keepdims=True)
    i1 = jnp.min(jnp.where(lg == m1, col, _EPAD), axis=1, keepdims=True)
    lg2 = jnp.where(col == i1, neg, lg)
    m2 = jnp.max(lg2, axis=1, keepdims=True)
    i2 = jnp.min(jnp.where(lg2 == m2, col, _EPAD), axis=1, keepdims=True)
    w1 = 1.0 / (1.0 + jnp.exp(m2 - m1))
    w2 = 1.0 - w1
    gates_ref[...] = jnp.where(col == i1, w1, 0.0) + jnp.where(col == i2, w2, 0.0)
    logits_ref[...] = lg


def _ffn_body(x_ref, gt_ref, w1_ref, b1_ref, w2_ref, b2_ref, out_ref):
    e = pl.program_id(0)
    f = pl.program_id(1)
    x = x_ref[...]
    h = jnp.dot(x, w1_ref[0], preferred_element_type=jnp.float32)
    h = jax.nn.gelu(h + b1_ref[0])
    y = jnp.dot(h, w2_ref[0], preferred_element_type=jnp.float32)
    y = y + jnp.where(f == 0, 1.0, 0.0) * b2_ref[0]
    contrib = gt_ref[0] * y
    first = (e == 0) & (f == 0)

    @pl.when(first)
    def _():
        out_ref[...] = contrib

    @pl.when(jnp.logical_not(first))
    def _():
        out_ref[...] = out_ref[...] + contrib


def kernel(x, Wg, bg, W1, b1, W2, b2):
    Bs, Ls, Ds = x.shape
    T = Bs * Ls
    E, Dff = W1.shape[0], W1.shape[2]
    x_flat = x.reshape(T, Ds)

    wg_pad = jnp.zeros((Ds, _EPAD), Wg.dtype).at[:, :E].set(Wg)
    bg_pad = jnp.full((1, _EPAD), -jnp.inf, bg.dtype).at[0, :E].set(bg)

    logits_pad, gates_pad = pl.pallas_call(
        _router_body,
        out_shape=(
            jax.ShapeDtypeStruct((T, _EPAD), jnp.float32),
            jax.ShapeDtypeStruct((T, _EPAD), jnp.float32),
        ),
    )(x_flat, wg_pad, bg_pad)

    logits = logits_pad[:, :E]
    gates_t = gates_pad[:, :E].T.reshape(E, T, 1)

    nf = Dff // _FBLK
    grid = (E, nf)
    out = pl.pallas_call(
        _ffn_body,
        grid=grid,
        in_specs=[
            pl.BlockSpec((T, Ds), lambda e, f: (0, 0)),
            pl.BlockSpec((1, T, 1), lambda e, f: (e, 0, 0)),
            pl.BlockSpec((1, Ds, _FBLK), lambda e, f: (e, 0, f)),
            pl.BlockSpec((1, 1, _FBLK), lambda e, f: (e, 0, f)),
            pl.BlockSpec((1, _FBLK, Ds), lambda e, f: (e, f, 0)),
            pl.BlockSpec((1, 1, Ds), lambda e, f: (e, 0, 0)),
        ],
        out_specs=pl.BlockSpec((T, Ds), lambda e, f: (0, 0)),
        out_shape=jax.ShapeDtypeStruct((T, Ds), jnp.float32),
    )(x_flat, gates_t, W1, b1.reshape(E, 1, Dff), W2, b2.reshape(E, 1, Ds))

    return out.reshape(Bs, Ls, Ds), logits
